# baseline (speedup 1.0000x reference)
import functools

import jax
import jax.numpy as jnp
from jax import lax
from jax.experimental import pallas as pl
from jax.experimental.pallas import tpu as pltpu
from jax.experimental.pallas import tpu_sc as plsc

NUM_BUCKETS = 1000
EMBED_DIM = 64
BATCH = 4096
L = 26

NCU = 2
NS = 16
ROWS = BATCH // (NCU * NS)
CHUNK = 64
NCHUNK = ROWS // CHUNK
NGRP = CHUNK // 16
TS = EMBED_DIM // 2 + 1
TW = NUM_BUCKETS * TS


def _qr_body(xf_hbm, wcat_hbm, out_hbm, wq_v, wr_v, xs_v, qb_v, rb_v, out_v2):
    c = lax.axis_index("c")
    s = lax.axis_index("s")
    rbase = (c * NS + s) * ROWS

    pltpu.sync_copy(wcat_hbm.at[pl.ds(0, TW)], wq_v)
    pltpu.sync_copy(wcat_hbm.at[pl.ds(TW, TW)], wr_v)

    iota = lax.iota(jnp.int32, 16)
    iota_l = iota * L

    def chunk_body(ch, _):
        row0 = rbase + ch * CHUNK
        pltpu.sync_copy(xf_hbm.at[pl.ds(row0 * L, CHUNK * L)], xs_v)

        def grp_body(g, _):
            goff = g * 16
            xbase = iota_l + g * (16 * L)

            @plsc.parallel_loop(0, L, unroll=13)
            def _prepass(l):
                xv = plsc.load_gather(xs_v, [xbase + l])
                xf32 = xv.astype(jnp.float32)
                qv = (xf32 * jnp.float32(0.001)).astype(jnp.int32)
                rv = xv - qv * NUM_BUCKETS
                qb_v[pl.ds(l * 16, 16)] = qv * TS
                rb_v[pl.ds(l * 16, 16)] = rv * TS

            for sub in range(4):
                zeros = tuple(jnp.zeros((16,), jnp.float32) for _ in range(16))

                @plsc.parallel_loop(0, L, unroll=13, carry=zeros)
                def _gather(l, accs):
                    qb = qb_v[pl.ds(l * 16, 16)]
                    rb = rb_v[pl.ds(l * 16, 16)]
                    new = list(accs)
                    for rr in range(4):
                        lane = jnp.full((16,), sub * 4 + rr, jnp.int32)
                        qs = jnp.take_along_axis(
                            qb, lane, axis=0, mode="promise_in_bounds") + iota
                        rs = jnp.take_along_axis(
                            rb, lane, axis=0, mode="promise_in_bounds") + iota
                        for h in range(2):
                            wq_h = wq_v.at[pl.ds(16 * h, TW - 16 * h)]
                            wr_h = wr_v.at[pl.ds(16 * h, TW - 16 * h)]
                            gq = plsc.load_gather(wq_h, [qs])
                            gr = plsc.load_gather(wr_h, [rs])
                            prod = plsc.bitcast(gq, jnp.bfloat16) * \
                                plsc.bitcast(gr, jnp.bfloat16)
                            pa, pb = plsc.unpack(
                                prod, format=plsc.PackFormat.INTERLEAVED)
                            k = rr * 4 + 2 * h
                            new[k] = new[k] + pa
                            new[k + 1] = new[k + 1] + pb
                    return tuple(new)

                for rr in range(4):
                    row = goff + sub * 4 + rr
                    out_v2[row, pl.ds(0, 16)] = _gather[rr * 4 + 0]
                    out_v2[row, pl.ds(32, 16)] = _gather[rr * 4 + 1]
                    out_v2[row, pl.ds(16, 16)] = _gather[rr * 4 + 2]
                    out_v2[row, pl.ds(48, 16)] = _gather[rr * 4 + 3]
            return 0

        lax.fori_loop(0, NGRP, grp_body, 0, unroll=False)
        pltpu.sync_copy(out_v2, out_hbm.at[pl.ds(row0, CHUNK), :])
        return 0

    lax.fori_loop(0, NCHUNK, chunk_body, 0, unroll=False)


@jax.jit
def _qr_embedding(x, Wq, Wr):
    def _pack(W):
        b = W.astype(jnp.bfloat16)
        pair = jnp.stack([b[:, :32], b[:, 32:]], axis=-1)
        w = jax.lax.bitcast_convert_type(pair, jnp.int32)
        return jnp.pad(w, ((0, 0), (0, TS - 32))).ravel()

    wcat = jnp.concatenate([_pack(Wq), _pack(Wr)])
    xf = x.ravel()
    mesh = plsc.VectorSubcoreMesh(
        core_axis_name="c", subcore_axis_name="s", num_cores=NCU)
    kern = functools.partial(
        pl.kernel,
        out_type=jax.ShapeDtypeStruct((BATCH, EMBED_DIM), jnp.float32),
        mesh=mesh,
        compiler_params=pltpu.CompilerParams(
            use_tc_tiling_on_sc=False, needs_layout_passes=False,
            skip_device_barrier=True
        ),
        scratch_types=[
            pltpu.VMEM((TW,), jnp.int32),
            pltpu.VMEM((TW,), jnp.int32),
            pltpu.VMEM((CHUNK * L,), jnp.int32),
            pltpu.VMEM((L * 16,), jnp.int32),
            pltpu.VMEM((L * 16,), jnp.int32),
            pltpu.VMEM((CHUNK, EMBED_DIM), jnp.float32),
        ],
    )(_qr_body)
    return kern(xf, wcat)


def kernel(x, Wq, Wr):
    return _qr_embedding(x.astype(jnp.int32), Wq, Wr)


# speedup vs baseline: 3.0571x; 3.0571x over previous
import functools

import jax
import jax.numpy as jnp
from jax import lax
from jax.experimental import pallas as pl
from jax.experimental.pallas import tpu as pltpu
from jax.experimental.pallas import tpu_sc as plsc

NUM_BUCKETS = 1000
EMBED_DIM = 64
BATCH = 4096
L = 26

NCU = 2
NS = 16
ROWS = BATCH // (NCU * NS)
CHUNK = 64
NCHUNK = ROWS // CHUNK
NGRP = CHUNK // 16
TS = EMBED_DIM // 2 + 1
TW = NUM_BUCKETS * TS


def _qr_body(xf_hbm, wcat_hbm, out_hbm, wq_v, wr_v, xs_v, qb_v, rb_v,
             out_v2, sem_q, sem_r):
    c = lax.axis_index("c")
    s = lax.axis_index("s")
    rbase = (c * NS + s) * ROWS

    cq = pltpu.make_async_copy(wcat_hbm.at[pl.ds(0, TW)], wq_v, sem_q)
    cr = pltpu.make_async_copy(wcat_hbm.at[pl.ds(TW, TW)], wr_v, sem_r)
    cq.start()
    cr.start()
    cq.wait()
    cr.wait()

    iota = lax.iota(jnp.int32, 16)
    iota_l = iota * L

    def chunk_body(ch, _):
        row0 = rbase + ch * CHUNK
        pltpu.sync_copy(xf_hbm.at[pl.ds(row0 * L, CHUNK * L)], xs_v)

        def grp_body(g, _):
            goff = g * 16
            xbase = iota_l + g * (16 * L)

            @plsc.parallel_loop(0, L, unroll=2)
            def _prepass(l):
                xv = plsc.load_gather(xs_v, [xbase + l])
                xf32 = xv.astype(jnp.float32)
                qv = (xf32 * jnp.float32(0.001)).astype(jnp.int32)
                rv = xv - qv * NUM_BUCKETS
                qb_v[pl.ds(l * 16, 16)] = qv * TS
                rb_v[pl.ds(l * 16, 16)] = rv * TS

            for sub in range(4):
                zeros = tuple(jnp.zeros((16,), jnp.float32) for _ in range(16))

                @plsc.parallel_loop(0, L, unroll=2, carry=zeros)
                def _gather(l, accs):
                    qb = qb_v[pl.ds(l * 16, 16)]
                    rb = rb_v[pl.ds(l * 16, 16)]
                    new = list(accs)
                    for rr in range(4):
                        lane = jnp.full((16,), sub * 4 + rr, jnp.int32)
                        qs = jnp.take_along_axis(
                            qb, lane, axis=0, mode="promise_in_bounds") + iota
                        rs = jnp.take_along_axis(
                            rb, lane, axis=0, mode="promise_in_bounds") + iota
                        for h in range(2):
                            wq_h = wq_v.at[pl.ds(16 * h, TW - 16 * h)]
                            wr_h = wr_v.at[pl.ds(16 * h, TW - 16 * h)]
                            gq = plsc.load_gather(wq_h, [qs])
                            gr = plsc.load_gather(wr_h, [rs])
                            prod = plsc.bitcast(gq, jnp.bfloat16) * \
                                plsc.bitcast(gr, jnp.bfloat16)
                            pa, pb = plsc.unpack(
                                prod, format=plsc.PackFormat.INTERLEAVED)
                            k = rr * 4 + 2 * h
                            new[k] = new[k] + pa
                            new[k + 1] = new[k + 1] + pb
                    return tuple(new)

                for rr in range(4):
                    row = goff + sub * 4 + rr
                    out_v2[row, pl.ds(0, 16)] = _gather[rr * 4 + 0]
                    out_v2[row, pl.ds(32, 16)] = _gather[rr * 4 + 1]
                    out_v2[row, pl.ds(16, 16)] = _gather[rr * 4 + 2]
                    out_v2[row, pl.ds(48, 16)] = _gather[rr * 4 + 3]
            return 0

        lax.fori_loop(0, NGRP, grp_body, 0, unroll=False)
        pltpu.sync_copy(out_v2, out_hbm.at[pl.ds(row0, CHUNK), :])
        return 0

    lax.fori_loop(0, NCHUNK, chunk_body, 0, unroll=False)


@jax.jit
def _qr_embedding(x, Wq, Wr):
    def _pack(W):
        b = W.astype(jnp.bfloat16)
        pair = jnp.stack([b[:, :32], b[:, 32:]], axis=-1)
        w = jax.lax.bitcast_convert_type(pair, jnp.int32)
        return jnp.pad(w, ((0, 0), (0, TS - 32))).ravel()

    wcat = jnp.concatenate([_pack(Wq), _pack(Wr)])
    xf = x.ravel()
    mesh = plsc.VectorSubcoreMesh(
        core_axis_name="c", subcore_axis_name="s", num_cores=NCU)
    kern = functools.partial(
        pl.kernel,
        out_type=jax.ShapeDtypeStruct((BATCH, EMBED_DIM), jnp.float32),
        mesh=mesh,
        compiler_params=pltpu.CompilerParams(
            use_tc_tiling_on_sc=False, needs_layout_passes=False,
            skip_device_barrier=True
        ),
        scratch_types=[
            pltpu.VMEM((TW,), jnp.int32),
            pltpu.VMEM((TW,), jnp.int32),
            pltpu.VMEM((CHUNK * L,), jnp.int32),
            pltpu.VMEM((L * 16,), jnp.int32),
            pltpu.VMEM((L * 16,), jnp.int32),
            pltpu.VMEM((CHUNK, EMBED_DIM), jnp.float32),
            pltpu.SemaphoreType.DMA,
            pltpu.SemaphoreType.DMA,
        ],
    )(_qr_body)
    return kern(xf, wcat)


def kernel(x, Wq, Wr):
    return _qr_embedding(x.astype(jnp.int32), Wq, Wr)


# speedup vs baseline: 3.1101x; 1.0173x over previous
import functools

import jax
import jax.numpy as jnp
from jax import lax
from jax.experimental import pallas as pl
from jax.experimental.pallas import tpu as pltpu
from jax.experimental.pallas import tpu_sc as plsc

NUM_BUCKETS = 1000
EMBED_DIM = 64
BATCH = 4096
L = 26

NCU = 2
NS = 16
ROWS = BATCH // (NCU * NS)
CHUNK = 64
NCHUNK = ROWS // CHUNK
NGRP = CHUNK // 16
TS = EMBED_DIM // 2 + 1
TW = NUM_BUCKETS * TS


def _qr_body(xf_hbm, wcat_hbm, out_hbm, wq_v, wr_v, xs_bufs, qb_v, rb_v,
             out_bufs, sem_q, sem_r, sems_x, sems_o):
    c = lax.axis_index("c")
    s = lax.axis_index("s")
    rbase = (c * NS + s) * ROWS

    cq = pltpu.make_async_copy(wcat_hbm.at[pl.ds(0, TW)], wq_v, sem_q)
    cr = pltpu.make_async_copy(wcat_hbm.at[pl.ds(TW, TW)], wr_v, sem_r)
    cq.start()
    cr.start()
    xcopies = []
    for ch in range(NCHUNK):
        h = pltpu.make_async_copy(
            xf_hbm.at[pl.ds((rbase + ch * CHUNK) * L, CHUNK * L)],
            xs_bufs[ch], sems_x[ch])
        h.start()
        xcopies.append(h)
    cq.wait()
    cr.wait()

    iota = lax.iota(jnp.int32, 16)
    iota_l = iota * L

    ocopies = []
    for ch in range(NCHUNK):
        row0 = rbase + ch * CHUNK
        xs_v = xs_bufs[ch]
        out_v2 = out_bufs[ch]
        xcopies[ch].wait()

        def grp_body(g, _):
            goff = g * 16
            xbase = iota_l + g * (16 * L)

            @plsc.parallel_loop(0, L, unroll=2)
            def _prepass(l):
                xv = plsc.load_gather(xs_v, [xbase + l])
                xf32 = xv.astype(jnp.float32)
                qv = (xf32 * jnp.float32(0.001)).astype(jnp.int32)
                rv = xv - qv * NUM_BUCKETS
                qb_v[pl.ds(l * 16, 16)] = qv * TS
                rb_v[pl.ds(l * 16, 16)] = rv * TS

            for sub in range(4):
                zeros = tuple(jnp.zeros((16,), jnp.float32) for _ in range(16))

                @plsc.parallel_loop(0, L, unroll=2, carry=zeros)
                def _gather(l, accs):
                    qb = qb_v[pl.ds(l * 16, 16)]
                    rb = rb_v[pl.ds(l * 16, 16)]
                    new = list(accs)
                    for rr in range(4):
                        lane = jnp.full((16,), sub * 4 + rr, jnp.int32)
                        qs = jnp.take_along_axis(
                            qb, lane, axis=0, mode="promise_in_bounds") + iota
                        rs = jnp.take_along_axis(
                            rb, lane, axis=0, mode="promise_in_bounds") + iota
                        for h in range(2):
                            wq_h = wq_v.at[pl.ds(16 * h, TW - 16 * h)]
                            wr_h = wr_v.at[pl.ds(16 * h, TW - 16 * h)]
                            gq = plsc.load_gather(wq_h, [qs])
                            gr = plsc.load_gather(wr_h, [rs])
                            prod = plsc.bitcast(gq, jnp.bfloat16) * \
                                plsc.bitcast(gr, jnp.bfloat16)
                            pa, pb = plsc.unpack(
                                prod, format=plsc.PackFormat.INTERLEAVED)
                            k = rr * 4 + 2 * h
                            new[k] = new[k] + pa
                            new[k + 1] = new[k + 1] + pb
                    return tuple(new)

                for rr in range(4):
                    row = goff + sub * 4 + rr
                    out_v2[row, pl.ds(0, 16)] = _gather[rr * 4 + 0]
                    out_v2[row, pl.ds(32, 16)] = _gather[rr * 4 + 1]
                    out_v2[row, pl.ds(16, 16)] = _gather[rr * 4 + 2]
                    out_v2[row, pl.ds(48, 16)] = _gather[rr * 4 + 3]
            return 0

        lax.fori_loop(0, NGRP, grp_body, 0, unroll=False)
        h = pltpu.make_async_copy(
            out_v2, out_hbm.at[pl.ds(row0, CHUNK), :], sems_o[ch])
        h.start()
        ocopies.append(h)

    for h in ocopies:
        h.wait()


@jax.jit
def _qr_embedding(x, Wq, Wr):
    def _pack(W):
        b = W.astype(jnp.bfloat16)
        pair = jnp.stack([b[:, :32], b[:, 32:]], axis=-1)
        w = jax.lax.bitcast_convert_type(pair, jnp.int32)
        return jnp.pad(w, ((0, 0), (0, TS - 32))).ravel()

    wcat = jnp.concatenate([_pack(Wq), _pack(Wr)])
    xf = x.ravel()
    mesh = plsc.VectorSubcoreMesh(
        core_axis_name="c", subcore_axis_name="s", num_cores=NCU)
    kern = functools.partial(
        pl.kernel,
        out_type=jax.ShapeDtypeStruct((BATCH, EMBED_DIM), jnp.float32),
        mesh=mesh,
        compiler_params=pltpu.CompilerParams(
            use_tc_tiling_on_sc=False, needs_layout_passes=False,
            skip_device_barrier=True
        ),
        scratch_types=[
            pltpu.VMEM((TW,), jnp.int32),
            pltpu.VMEM((TW,), jnp.int32),
            [pltpu.VMEM((CHUNK * L,), jnp.int32) for _ in range(NCHUNK)],
            pltpu.VMEM((L * 16,), jnp.int32),
            pltpu.VMEM((L * 16,), jnp.int32),
            [pltpu.VMEM((CHUNK, EMBED_DIM), jnp.float32)
             for _ in range(NCHUNK)],
            pltpu.SemaphoreType.DMA,
            pltpu.SemaphoreType.DMA,
            [pltpu.SemaphoreType.DMA for _ in range(NCHUNK)],
            [pltpu.SemaphoreType.DMA for _ in range(NCHUNK)],
        ],
    )(_qr_body)
    return kern(xf, wcat)


def kernel(x, Wq, Wr):
    return _qr_embedding(x.astype(jnp.int32), Wq, Wr)
